# Initial kernel scaffold; baseline (speedup 1.0000x reference)
import jax
import jax.numpy as jnp
from jax.experimental import pallas as pl


def kernel(x, runningmean, runningvar):
    raise NotImplementedError("write your pallas kernel here")


# speedup vs baseline: 3.2165x; 3.2165x over previous
import functools

import jax
import jax.numpy as jnp
from jax.experimental import pallas as pl
from jax.experimental.pallas import tpu as pltpu

_EPS = 1e-5
_MOMENTUM = 0.9
_VMEM_LIMIT = 48 * 1024 * 1024


def _bn_train_kernel(x_ref, rm_ref, rv_ref, o_ref, nrm_ref, nrv_ref,
                     *, inv_count):
    x = x_ref[...]
    s = jnp.sum(x, axis=0)
    sq = jnp.sum(x * x, axis=0)
    mean = jnp.sum(s, axis=1, keepdims=True) * inv_count
    mean_sq = jnp.sum(sq, axis=1, keepdims=True) * inv_count
    var = jnp.maximum(mean_sq - mean * mean, 0.0)
    inv_std = jax.lax.rsqrt(var + _EPS)
    scale = inv_std[None]
    bias = (-mean * inv_std)[None]
    o_ref[...] = x * scale + bias
    nrm_ref[...] = (1.0 - _MOMENTUM) * mean + _MOMENTUM * rm_ref[...]
    nrv_ref[...] = (1.0 - _MOMENTUM) * var + _MOMENTUM * rv_ref[...]


@jax.jit
def kernel(x, runningmean, runningvar):
    B, C, H, W = x.shape
    hw = H * W
    x3 = x.reshape(B, C, hw)

    c_tile = C
    target = max(1, (4 * 1024 * 1024) // (B * hw * 4))
    while c_tile > target and c_tile % 2 == 0:
        c_tile //= 2
    if C % c_tile:
        c_tile = C
    grid = (C // c_tile,)

    rm2 = runningmean.astype(jnp.float32).reshape(C, 1)
    rv2 = runningvar.astype(jnp.float32).reshape(C, 1)

    out, nrm, nrv = pl.pallas_call(
        functools.partial(_bn_train_kernel, inv_count=1.0 / (B * hw)),
        out_shape=(jax.ShapeDtypeStruct((B, C, hw), jnp.float32),
                   jax.ShapeDtypeStruct((C, 1), jnp.float32),
                   jax.ShapeDtypeStruct((C, 1), jnp.float32)),
        grid=grid,
        in_specs=[
            pl.BlockSpec((B, c_tile, hw), lambda c: (0, c, 0)),
            pl.BlockSpec((c_tile, 1), lambda c: (c, 0)),
            pl.BlockSpec((c_tile, 1), lambda c: (c, 0)),
        ],
        out_specs=(
            pl.BlockSpec((B, c_tile, hw), lambda c: (0, c, 0)),
            pl.BlockSpec((c_tile, 1), lambda c: (c, 0)),
            pl.BlockSpec((c_tile, 1), lambda c: (c, 0)),
        ),
        compiler_params=pltpu.CompilerParams(
            dimension_semantics=("parallel",),
            vmem_limit_bytes=_VMEM_LIMIT,
        ),
    )(x3, rm2, rv2)

    return out.reshape(B, C, H, W), nrm.reshape(C), nrv.reshape(C)
